# baseline (speedup 1.0000x reference)
import functools

import jax
import jax.numpy as jnp
from jax import lax
from jax.experimental import pallas as pl
from jax.experimental.pallas import tpu as pltpu
from jax.experimental.pallas import tpu_sc as plsc

EMBED = 64
NC, NS = 2, 16
NW = NC * NS

B_TOTAL = 4096 * 200
B_PER_W = B_TOTAL // NW
GATHER = 128
K = 4
CHUNK = K * GATHER
NCHUNK = B_PER_W // CHUNK
NPAIR = NCHUNK // 2
ROWS_PER_W = B_PER_W // GATHER


def _sc_embed(idx2d, table):
    mesh = plsc.VectorSubcoreMesh(core_axis_name="c", subcore_axis_name="s")

    @functools.partial(
        pl.kernel,
        mesh=mesh,
        out_type=jax.ShapeDtypeStruct((B_TOTAL, EMBED), jnp.float32),
        scratch_types=[
            pltpu.VMEM((ROWS_PER_W, GATHER), jnp.int32),
            pltpu.VMEM((CHUNK, EMBED), jnp.float32),
            pltpu.VMEM((CHUNK, EMBED), jnp.float32),
            pltpu.SemaphoreType.DMA,
            pltpu.SemaphoreType.DMA,
            pltpu.SemaphoreType.DMA,
            pltpu.SemaphoreType.DMA,
        ],
        compiler_params=pltpu.CompilerParams(use_tc_tiling_on_sc=False),
    )
    def body(idx_hbm, table_hbm, out_hbm, idx_all, rows0, rows1,
             gsem0, gsem1, osem0, osem1):
        wid = lax.axis_index("s") * NC + lax.axis_index("c")
        row_base = wid * ROWS_PER_W
        out_base = wid * B_PER_W

        rows_bufs = (rows0, rows1)
        gsems = (gsem0, gsem1)
        osems = (osem0, osem1)

        pltpu.sync_copy(idx_hbm.at[pl.ds(row_base, ROWS_PER_W)], idx_all)

        def fire_gathers(g, b):
            for j in range(K):
                pltpu.async_copy(
                    table_hbm.at[idx_all.at[g * K + j]],
                    rows_bufs[b].at[pl.ds(j * GATHER, GATHER)],
                    gsems[b],
                )

        def drain_gathers(b):
            for j in range(K):
                pltpu.make_async_copy(
                    table_hbm.at[idx_all.at[j]],
                    rows_bufs[b].at[pl.ds(j * GATHER, GATHER)],
                    gsems[b],
                ).wait()

        def fire_store(g, b):
            pltpu.async_copy(
                rows_bufs[b], out_hbm.at[pl.ds(out_base + g * CHUNK, CHUNK)], osems[b]
            )

        def wait_store(b):
            pltpu.make_async_copy(
                rows_bufs[b], out_hbm.at[pl.ds(out_base, CHUNK)], osems[b]
            ).wait()

        fire_gathers(0, 0)
        fire_gathers(1, 1)

        def step(t, carry):
            g = t * 2
            drain_gathers(0)
            fire_store(g, 0)
            drain_gathers(1)
            fire_store(g + 1, 1)
            wait_store(0)
            fire_gathers(g + 2, 0)
            wait_store(1)
            fire_gathers(g + 3, 1)
            return carry

        lax.fori_loop(0, NPAIR - 1, step, 0)

        g = NCHUNK - 2
        drain_gathers(0)
        fire_store(g, 0)
        drain_gathers(1)
        fire_store(g + 1, 1)
        wait_store(0)
        wait_store(1)

    return body(idx2d, table)


def kernel(words_seq, table):
    idx2d = words_seq.astype(jnp.int32).reshape(B_TOTAL // GATHER, GATHER)
    out = _sc_embed(idx2d, table.astype(jnp.float32))
    return out.reshape(words_seq.shape[0], words_seq.shape[1], EMBED)
